# baseline (speedup 1.0000x reference)
import functools

import jax
import jax.numpy as jnp
from jax.experimental import pallas as pl
from jax.experimental.pallas import tpu as pltpu
from jax.experimental.pallas import tpu_sc as plsc

NUM_HEADS = 16
NUM_BUCKETS = 32
QLEN = 2048
KLEN = 2048
WS = 4096
BW = 384
RB = 8
NBLK = QLEN // 2 // RB
NCHUNK = KLEN // 128
INFLIGHT = 4

_THRESHOLDS = (1, 2, 3, 4, 5, 6, 7, 8, 12, 16, 23, 32, 46, 64, 91)


def _f_body(tblt_ref, out_ref):
    tblt = tblt_ref[...]
    m = jax.lax.broadcasted_iota(jnp.int32, (1, WS), 1)
    d = m - (QLEN - 1)
    x = jnp.abs(d)
    g = jnp.zeros((1, WS), jnp.int32)
    for t in _THRESHOLDS:
        g = g + (x >= t).astype(jnp.int32)
    bucket = g + 16 * (d > 0).astype(jnp.int32)
    f = jnp.zeros((NUM_HEADS, WS), jnp.float32)
    for b in range(NUM_BUCKETS):
        mask = (bucket == b).astype(jnp.float32)
        f = f + tblt[:, b:b + 1] * mask
    out_ref[...] = f


def _c_body(f_ref, out_ref):
    f = f_ref[...]
    cneg = jnp.broadcast_to(f[:, 0:1, None], (NUM_HEADS, RB, 128))
    cpos = jnp.broadcast_to(f[:, 2144:2145, None], (NUM_HEADS, RB, 128))
    out_ref[:, 0:RB, :] = cneg
    out_ref[:, RB:2 * RB, :] = cpos


_f_call = pl.pallas_call(
    _f_body,
    out_shape=jax.ShapeDtypeStruct((NUM_HEADS, WS), jnp.float32),
)

_c_call = pl.pallas_call(
    _c_body,
    out_shape=jax.ShapeDtypeStruct((NUM_HEADS, 2 * RB, 128), jnp.float32),
)


@functools.lru_cache(maxsize=1)
def _make_expand():
    mesh = plsc.VectorSubcoreMesh(
        core_axis_name="c", subcore_axis_name="s", num_cores=2, num_subcores=16
    )

    @functools.partial(
        pl.kernel,
        out_type=jax.ShapeDtypeStruct((NUM_HEADS * QLEN, KLEN), jnp.float32),
        mesh=mesh,
        scratch_types=[
            pltpu.VMEM((WS,), jnp.float32),
            pltpu.VMEM((RB, BW), jnp.float32),
            pltpu.VMEM_SHARED((8, BW, BW), jnp.float32),
            pltpu.SemaphoreType.DMA,
        ],
    )
    def expand(f_hbm, cb_hbm, out_hbm, fv, st, t_sh, sem):
        c = jax.lax.axis_index("c")
        sid = jax.lax.axis_index("s")
        wid = c * 16 + sid
        h = wid // 2
        hh = h % 8
        half = wid % 2
        pltpu.sync_copy(f_hbm.at[h], fv)

        wb = half * (BW // 2)
        for tb in range(BW // 2 // RB):
            w0t = wb + tb * RB

            def copy_chunk(cc, carry, w0t=w0t):
                base = (QLEN - 1) - w0t + cc * 16
                col = cc * 16
                for ri in range(RB):
                    st[ri, pl.ds(col, 16)] = fv[pl.ds(base - ri, 16)]
                return carry

            jax.lax.fori_loop(0, BW // 16, copy_chunk, 0, unroll=2)
            pltpu.sync_copy(st, t_sh.at[hh, pl.ds(w0t, RB), :])

        plsc.subcore_barrier()

        rbase = wid * (QLEN // 2)

        def blk_wait(blk):
            row0 = pl.multiple_of(rbase + blk * RB, RB)
            ref = out_hbm.at[pl.ds(row0, RB), :]
            pltpu.make_async_copy(ref, ref, sem).wait()

        def step(blk, carry):
            q0 = half * (QLEN // 2) + blk * RB
            cm = jnp.maximum(q0 - 90, 0) & -128
            cwin = jnp.minimum(cm, KLEN - BW)
            w0 = q0 - cwin
            row0 = pl.multiple_of(rbase + blk * RB, RB)
            for bi in range(3):
                src_b = t_sh.at[hh, pl.ds(pl.multiple_of(w0, RB), RB),
                                pl.ds(bi * 128, 128)]
                dst_b = out_hbm.at[
                    pl.ds(row0, RB),
                    pl.ds(pl.multiple_of(cwin + bi * 128, 128), 128)]
                pltpu.make_async_copy(src_b, dst_b, sem).start()
            for j in range(NCHUNK - 3):
                colj = jnp.where(j * 128 >= cwin, j * 128 + BW, j * 128)
                rsel = jnp.where(colj < cwin, 0, RB)
                src = cb_hbm.at[h, pl.ds(pl.multiple_of(rsel, RB), RB), :]
                dst = out_hbm.at[pl.ds(row0, RB),
                                 pl.ds(pl.multiple_of(colj, 128), 128)]
                pltpu.make_async_copy(src, dst, sem).start()
            return carry

        for blk in range(INFLIGHT):
            step(blk, 0)

        def pipelined(i, carry):
            blk_wait(i - INFLIGHT)
            step(i, carry)
            return carry

        jax.lax.fori_loop(INFLIGHT, NBLK, pipelined, 0)
        for blk in range(NBLK - INFLIGHT, NBLK):
            blk_wait(blk)

    return expand


@jax.jit
def _impl(table):
    f = _f_call(table.T)
    cb = _c_call(f)
    flat = _make_expand()(f, cb)
    return flat.reshape(1, NUM_HEADS, QLEN, KLEN)


def kernel(query_length, key_length, table):
    return _impl(table)


# speedup vs baseline: 36.6727x; 36.6727x over previous
import functools

import jax
import jax.numpy as jnp
from jax.experimental import pallas as pl
from jax.experimental.pallas import tpu as pltpu
from jax.experimental.pallas import tpu_sc as plsc

NUM_HEADS = 16
NUM_BUCKETS = 32
QLEN = 2048
KLEN = 2048
WS = 4096
BW = 384
TROWS = 400
RB = 8
NBLK = QLEN // 2 // RB
NCHUNK = KLEN // 128
INFLIGHT = 4

_THRESHOLDS = (1, 2, 3, 4, 5, 6, 7, 8, 12, 16, 23, 32, 46, 64, 91)


def _f_body(tblt_ref, out_ref):
    tblt = tblt_ref[...]
    m = jax.lax.broadcasted_iota(jnp.int32, (1, WS), 1)
    d = m - (QLEN - 1)
    x = jnp.abs(d)
    g = jnp.zeros((1, WS), jnp.int32)
    for t in _THRESHOLDS:
        g = g + (x >= t).astype(jnp.int32)
    bucket = g + 16 * (d > 0).astype(jnp.int32)
    f = jnp.zeros((NUM_HEADS, WS), jnp.float32)
    for b in range(NUM_BUCKETS):
        mask = (bucket == b).astype(jnp.float32)
        f = f + tblt[:, b:b + 1] * mask
    out_ref[...] = f


_f_call = pl.pallas_call(
    _f_body,
    out_shape=jax.ShapeDtypeStruct((NUM_HEADS, WS), jnp.float32),
)


@functools.lru_cache(maxsize=1)
def _make_expand():
    mesh = plsc.VectorSubcoreMesh(
        core_axis_name="c", subcore_axis_name="s", num_cores=2, num_subcores=16
    )

    @functools.partial(
        pl.kernel,
        out_type=jax.ShapeDtypeStruct((NUM_HEADS * QLEN, KLEN), jnp.float32),
        mesh=mesh,
        scratch_types=[
            pltpu.VMEM((WS,), jnp.float32),
            pltpu.VMEM((RB, BW), jnp.float32),
            pltpu.VMEM_SHARED((8, TROWS, BW), jnp.float32),
            pltpu.SemaphoreType.DMA,
        ],
    )
    def expand(f_hbm, out_hbm, fv, st, t_sh, sem):
        c = jax.lax.axis_index("c")
        sid = jax.lax.axis_index("s")
        wid = c * 16 + sid
        h = wid // 2
        hh = h % 8
        half = wid % 2
        pltpu.sync_copy(f_hbm.at[h], fv)

        wb = half * (BW // 2)
        for tb in range(BW // 2 // RB):
            w0t = wb + tb * RB

            def copy_chunk(cc, carry, w0t=w0t):
                base = (QLEN - 1) - w0t + cc * 16
                col = cc * 16
                for ri in range(RB):
                    st[ri, pl.ds(col, 16)] = fv[pl.ds(base - ri, 16)]
                return carry

            jax.lax.fori_loop(0, BW // 16, copy_chunk, 0, unroll=2)
            pltpu.sync_copy(st, t_sh.at[hh, pl.ds(w0t, RB), :])

        @pl.when(half == 1)
        def _():
            cneg = fv[pl.ds(0, 16)]
            cpos = fv[pl.ds(2144, 16)]
            for ri in range(RB):
                for cc in range(BW // 16):
                    st[ri, pl.ds(cc * 16, 16)] = cneg
            pltpu.sync_copy(st, t_sh.at[hh, pl.ds(BW, RB), :])
            for ri in range(RB):
                for cc in range(BW // 16):
                    st[ri, pl.ds(cc * 16, 16)] = cpos
            pltpu.sync_copy(st, t_sh.at[hh, pl.ds(BW + RB, RB), :])

        plsc.subcore_barrier()

        rbase = wid * (QLEN // 2)

        def blk_wait(blk):
            row0 = pl.multiple_of(rbase + blk * RB, RB)
            ref = out_hbm.at[pl.ds(row0, RB), :]
            pltpu.make_async_copy(ref, ref, sem).wait()

        def step(blk, carry):
            q0 = half * (QLEN // 2) + blk * RB
            cm = jnp.maximum(q0 - 90, 0) & -128
            cwin = jnp.minimum(cm, KLEN - BW)
            w0 = q0 - cwin
            row0 = pl.multiple_of(rbase + blk * RB, RB)
            for cb in range(NCHUNK):
                col = cb * 128
                in_band = jnp.logical_and(col >= cwin, col < cwin + BW)
                rsel = jnp.where(in_band, w0,
                                 jnp.where(col < cwin, BW, BW + RB))
                csel = jnp.where(in_band, col - cwin, 0)
                src = t_sh.at[hh,
                              pl.ds(pl.multiple_of(rsel, RB), RB),
                              pl.ds(pl.multiple_of(csel, 128), 128)]
                dst = out_hbm.at[pl.ds(row0, RB), pl.ds(col, 128)]
                pltpu.make_async_copy(src, dst, sem).start()
            return carry

        for blk in range(INFLIGHT):
            step(blk, 0)

        def pipelined(i, carry):
            blk_wait(i - INFLIGHT)
            step(i, carry)
            return carry

        jax.lax.fori_loop(INFLIGHT, NBLK, pipelined, 0)
        for blk in range(NBLK - INFLIGHT, NBLK):
            blk_wait(blk)

    return expand


@jax.jit
def _impl(table):
    f = _f_call(table.T)
    flat = _make_expand()(f)
    return flat.reshape(1, NUM_HEADS, QLEN, KLEN)


def kernel(query_length, key_length, table):
    return _impl(table)
